# baseline (speedup 1.0000x reference)
import functools

import jax
import jax.numpy as jnp
from jax import lax
from jax.experimental import pallas as pl
from jax.experimental.pallas import tpu as pltpu
from jax.experimental.pallas import tpu_sc as plsc

F32 = jnp.float32
I32 = jnp.int32
BF16 = jnp.bfloat16

N = 10000
E = 320000
H = 128
TD = 64
L = 16
NPAD = 10240
NSC = 2
NSUB = 16
NW = NSC * NSUB
CH = 80
RPT = NPAD // NSUB

_mesh = plsc.VectorSubcoreMesh(core_axis_name="c", subcore_axis_name="s")
_NCHUNK = (E // NW) // CH
_NCPAD = 128


def _wid():
    return lax.axis_index("s") * NSC + lax.axis_index("c")


@functools.partial(
    pl.kernel, mesh=_mesh,
    out_type=jax.ShapeDtypeStruct((NSC, NPAD, H), F32),
    scratch_types=[
        pltpu.VMEM((CH,), I32),
        pltpu.VMEM((CH,), I32),
        pltpu.VMEM((CH, H), F32),
        pltpu.VMEM_SHARED((NPAD, H), F32),
    ],
)
def _sc_deg(dst_hbm, ones_hbm, zer_hbm, out_hbm, dstv0, dstv1, onesv, acc):
    c = lax.axis_index("c")
    s = lax.axis_index("s")
    wid = _wid()
    dstv = (dstv0, dstv1)
    pltpu.sync_copy(zer_hbm, acc.at[pl.ds(s * RPT, RPT), :])
    pltpu.sync_copy(ones_hbm, onesv)
    plsc.subcore_barrier()
    base = wid * (E // NW)

    def issue(ci, b):
        pltpu.sync_copy(dst_hbm.at[pl.ds(base + ci * CH, CH)], dstv[b])

    def drain(b):
        pltpu.sync_copy(onesv, acc.at[dstv[b]], add=True)

    issue(0, 0)

    def body(ci, _):
        issue(2 * ci + 1, 1)
        drain(0)
        issue(2 * ci + 2, 0)
        drain(1)
        return 0

    lax.fori_loop(0, (_NCHUNK - 1) // 2, body, 0)
    drain(0)
    plsc.subcore_barrier()
    pltpu.sync_copy(acc.at[pl.ds(s * RPT, RPT), :],
                    out_hbm.at[c, pl.ds(s * RPT, RPT), :])


@functools.partial(
    pl.kernel, mesh=_mesh,
    out_type=jax.ShapeDtypeStruct((NSC, NPAD, H), F32),
    scratch_types=[
        pltpu.VMEM((E // NW,), I32),
        pltpu.VMEM((CH,), I32),
        pltpu.VMEM((CH,), I32),
        pltpu.VMEM((CH, H), F32),
        pltpu.VMEM((CH, H), F32),
        pltpu.VMEM_SHARED((NPAD, H), F32),
        pltpu.SemaphoreType.DMA,
        pltpu.SemaphoreType.DMA,
    ],
)
def _sc_segsum(g_hbm, src_hbm, dst_hbm, zer_hbm, out_hbm,
               srcall, dstv0, dstv1, rows0, rows1, acc, sem0, sem1):
    c = lax.axis_index("c")
    s = lax.axis_index("s")
    wid = _wid()
    dstv = (dstv0, dstv1)
    rows = (rows0, rows1)
    sem = (sem0, sem1)
    epw = E // NW
    base = wid * epw
    pltpu.sync_copy(zer_hbm, acc.at[pl.ds(s * RPT, RPT), :])
    pltpu.sync_copy(src_hbm.at[pl.ds(base, epw)], srcall)
    plsc.subcore_barrier()

    def issue(ci, b):
        pltpu.sync_copy(dst_hbm.at[pl.ds(base + ci * CH, CH)], dstv[b])
        pltpu.async_copy(g_hbm.at[srcall.at[pl.ds(ci * CH, CH)]],
                         rows[b], sem[b])

    def drain(ci, b):
        pltpu.make_async_copy(g_hbm.at[srcall.at[pl.ds(ci * CH, CH)]],
                              rows[b], sem[b]).wait()
        pltpu.sync_copy(rows[b], acc.at[dstv[b]], add=True)

    issue(0, 0)

    def body(ci, _):
        issue(2 * ci + 1, 1)
        drain(2 * ci, 0)
        issue(2 * ci + 2, 0)
        drain(2 * ci + 1, 1)
        return 0

    lax.fori_loop(0, (_NCHUNK - 1) // 2, body, 0)
    drain(_NCHUNK - 1, 0)
    plsc.subcore_barrier()
    pltpu.sync_copy(acc.at[pl.ds(s * RPT, RPT), :],
                    out_hbm.at[c, pl.ds(s * RPT, RPT), :])


@functools.partial(
    pl.kernel, mesh=_mesh,
    out_type=jax.ShapeDtypeStruct((NPAD * L, H), F32),
    scratch_types=[
        pltpu.VMEM((CH,), I32),
        pltpu.VMEM((CH,), I32),
        pltpu.VMEM((CH, H), F32),
        pltpu.VMEM((CH, H), F32),
        pltpu.SemaphoreType.DMA,
        pltpu.SemaphoreType.DMA,
    ],
)
def _sc_embgather(tab_hbm, idx_hbm, out_hbm, idxv0, idxv1, rows0, rows1,
                  sem0, sem1):
    wid = _wid()
    per = (NPAD * L) // NW
    base = wid * per
    nchunk = per // CH
    idxv = (idxv0, idxv1)
    rows = (rows0, rows1)
    sem = (sem0, sem1)

    def issue(ci, b):
        off = base + ci * CH
        pltpu.sync_copy(idx_hbm.at[pl.ds(off, CH)], idxv[b])
        pltpu.async_copy(tab_hbm.at[idxv[b]], rows[b], sem[b])

    def drain(ci, b):
        pltpu.make_async_copy(tab_hbm.at[idxv[b]], rows[b], sem[b]).wait()
        off = base + ci * CH
        pltpu.sync_copy(rows[b], out_hbm.at[pl.ds(off, CH), :])

    issue(0, 0)

    def body(ci, _):
        issue(2 * ci + 1, 1)
        drain(2 * ci, 0)
        issue(2 * ci + 2, 0)
        drain(2 * ci + 1, 1)
        return 0

    lax.fori_loop(0, nchunk // 2 - 1, body, 0)
    issue(nchunk - 1, 1)
    drain(nchunk - 2, 0)
    drain(nchunk - 1, 1)


@functools.partial(
    pl.kernel, mesh=_mesh,
    out_type=jax.ShapeDtypeStruct((2, NPAD, H), F32),
    scratch_types=[
        pltpu.VMEM((NPAD // NW,), F32),
        pltpu.VMEM((NPAD // NW,), F32),
        pltpu.VMEM((NPAD // NW,), F32),
        pltpu.VMEM((NPAD // NW,), F32),
        pltpu.VMEM((CH,), I32),
        pltpu.VMEM((CH,), I32),
        pltpu.VMEM((CH, H), F32),
        pltpu.SemaphoreType.DMA,
    ],
)
def _sc_boxgather(fmT_hbm, px_hbm, py_hbm, cx_hbm, cy_hbm, out_hbm,
                  pxv, pyv, cxv, cyv, idx0, idx1, rows, sem):
    wid = _wid()
    per = NPAD // NW
    base = wid * per
    pltpu.sync_copy(px_hbm.at[pl.ds(base, per)], pxv)
    pltpu.sync_copy(py_hbm.at[pl.ds(base, per)], pyv)
    pltpu.sync_copy(cx_hbm.at[pl.ds(base, per)], cxv)
    pltpu.sync_copy(cy_hbm.at[pl.ds(base, per)], cyv)

    def body(ci, _):
        off = base + ci * CH
        for j in range(CH // 16):
            cs = pl.ds(ci * CH + j * 16, 16)
            sl = pl.ds(j * 16, 16)
            vx = pxv[cs]
            vy = pyv[cs]
            wx = cxv[cs]
            wy = cyv[cs]
            for snum, tt in ((0, -0.25), (1, 0.25)):
                qx = jnp.minimum(jnp.maximum(vx + tt * wx, 0.0), 1.0) * 64.0
                qy = jnp.minimum(jnp.maximum(vy + tt * wy, 0.0), 1.0) * 64.0
                ix = jnp.minimum(jnp.maximum(qx.astype(I32), 0), 63)
                iy = jnp.minimum(jnp.maximum(qy.astype(I32), 0), 63)
                lin = iy * 64 + ix
                if snum == 0:
                    idx0[sl] = lin
                else:
                    idx1[sl] = lin
        pltpu.async_copy(fmT_hbm.at[idx0], rows, sem).wait()
        pltpu.sync_copy(rows, out_hbm.at[0, pl.ds(off, CH), :])
        pltpu.async_copy(fmT_hbm.at[idx1], rows, sem).wait()
        pltpu.sync_copy(rows, out_hbm.at[1, pl.ds(off, CH), :])
        return 0

    lax.fori_loop(0, per // CH, body, 0)


EFW = 3 * H
NSLAB = 5
ES = E // NSLAB
ESCH = 80


def _make_edgefeat(es, ch=CH):
    nchunk = (es // NW) // ch

    @functools.partial(
        pl.kernel, mesh=_mesh,
        out_type=jax.ShapeDtypeStruct((es, EFW), F32),
        scratch_types=[
            pltpu.VMEM((es // NW,), I32),
            pltpu.VMEM((es // NW,), I32),
            pltpu.VMEM((ch, EFW), F32),
            pltpu.VMEM((ch, EFW), F32),
            pltpu.VMEM((ch, EFW), F32),
            pltpu.VMEM((ch, EFW), F32),
            pltpu.SemaphoreType.DMA,
            pltpu.SemaphoreType.DMA,
            pltpu.SemaphoreType.DMA,
            pltpu.SemaphoreType.DMA,
        ],
    )
    def _ef(stab_hbm, dtab_hbm, src_hbm, dst_hbm, out_hbm,
            srcall, dstall, ar0, br0, ar1, br1,
            sa0, sb0, sa1, sb1):
        wid = _wid()
        epw = es // NW
        base = wid * epw
        ar = (ar0, ar1)
        br = (br0, br1)
        sa = (sa0, sa1)
        sb = (sb0, sb1)
        pltpu.sync_copy(src_hbm.at[pl.ds(base, epw)], srcall)
        pltpu.sync_copy(dst_hbm.at[pl.ds(base, epw)], dstall)

        def issue(ci, b):
            sl = pl.ds(ci * ch, ch)
            pltpu.async_copy(stab_hbm.at[srcall.at[sl]], ar[b], sa[b])
            pltpu.async_copy(dtab_hbm.at[dstall.at[sl]], br[b], sb[b])

        def drain(ci, b):
            sl = pl.ds(ci * ch, ch)
            pltpu.make_async_copy(stab_hbm.at[srcall.at[sl]], ar[b], sa[b]).wait()
            pltpu.make_async_copy(dtab_hbm.at[dstall.at[sl]], br[b], sb[b]).wait()
            arb = ar[b]
            brb = br[b]

            def rowbody(i, _):
                for j in range(EFW // 16):
                    sl = pl.ds(j * 16, 16)
                    arb[i, sl] = jnp.maximum(arb[i, sl] + brb[i, sl], 0.0)
                return 0

            lax.fori_loop(0, ch, rowbody, 0)
            off = base + ci * ch
            pltpu.sync_copy(arb, out_hbm.at[pl.ds(off, ch), :])

        issue(0, 0)

        def body(ci, _):
            issue(2 * ci + 1, 1)
            drain(2 * ci, 0)
            issue(2 * ci + 2, 0)
            drain(2 * ci + 1, 1)
            return 0

        lax.fori_loop(0, (nchunk - 1) // 2, body, 0)
        drain(nchunk - 1, 0)

    return _ef


_sc_edgefeat = _make_edgefeat(ES, ESCH)


def _dinv_body(p_ref, o_ref):
    s = p_ref[0] + p_ref[1]
    o_ref[...] = lax.rsqrt(s[:, 0:1] + 1.0)


def _tc_dinv(parts):
    BB = 2048
    return pl.pallas_call(
        _dinv_body,
        grid=(NPAD // BB,),
        in_specs=[pl.BlockSpec((NSC, BB, H), lambda i: (0, i, 0))],
        out_specs=pl.BlockSpec((BB, 1), lambda i: (i, 0)),
        out_shape=jax.ShapeDtypeStruct((NPAD, 1), F32),
    )(parts)


BLN = 1024


def _mm_scale_body(x_ref, w_ref, dv_ref, o_ref):
    o_ref[...] = jnp.dot(x_ref[...], w_ref[...],
                         preferred_element_type=F32) * dv_ref[...]


def _tc_mm_scale(x, W, dinv):
    n = x.shape[0]
    return pl.pallas_call(
        _mm_scale_body,
        grid=(n // BLN,),
        in_specs=[
            pl.BlockSpec((BLN, x.shape[1]), lambda i: (i, 0)),
            pl.BlockSpec(W.shape, lambda i: (0, 0)),
            pl.BlockSpec((BLN, 1), lambda i: (i, 0)),
        ],
        out_specs=pl.BlockSpec((BLN, H), lambda i: (i, 0)),
        out_shape=jax.ShapeDtypeStruct((n, H), F32),
    )(x, W, dinv)


def _comb_mm_body(p0_ref, p1_ref, g_ref, dv_ref, b_ref, w_ref, o_ref):
    t = jnp.maximum(dv_ref[...] * (p0_ref[...] + p1_ref[...] + g_ref[...])
                    + b_ref[...], 0.0)
    o_ref[...] = jnp.dot(t, w_ref[...], preferred_element_type=F32) * dv_ref[...]


def _tc_combine_mm_scale(p0, p1, g, dinv, b, W):
    return pl.pallas_call(
        _comb_mm_body,
        grid=(NPAD // BLN,),
        in_specs=[
            pl.BlockSpec((BLN, H), lambda i: (i, 0)),
            pl.BlockSpec((BLN, H), lambda i: (i, 0)),
            pl.BlockSpec((BLN, H), lambda i: (i, 0)),
            pl.BlockSpec((BLN, 1), lambda i: (i, 0)),
            pl.BlockSpec((1, H), lambda i: (0, 0)),
            pl.BlockSpec((H, H), lambda i: (0, 0)),
        ],
        out_specs=pl.BlockSpec((BLN, H), lambda i: (i, 0)),
        out_shape=jax.ShapeDtypeStruct((NPAD, H), F32),
    )(p0, p1, g, dinv, b.reshape(1, H), W)


def _comb_proj_body(p0_ref, p1_ref, g_ref, dv_ref, b_ref, wa_ref, wb_ref,
                    bb_ref, a_ref, bo_ref):
    t = jnp.maximum(dv_ref[...] * (p0_ref[...] + p1_ref[...] + g_ref[...])
                    + b_ref[...], 0.0)
    a_ref[...] = jnp.dot(t, wa_ref[...], preferred_element_type=F32)
    bo_ref[...] = jnp.dot(t, wb_ref[...], preferred_element_type=F32) + bb_ref[...]


def _tc_combine_proj(p0, p1, g, dinv, b, Wa, Wb, bb):
    return pl.pallas_call(
        _comb_proj_body,
        grid=(NPAD // BLN,),
        in_specs=[
            pl.BlockSpec((BLN, H), lambda i: (i, 0)),
            pl.BlockSpec((BLN, H), lambda i: (i, 0)),
            pl.BlockSpec((BLN, H), lambda i: (i, 0)),
            pl.BlockSpec((BLN, 1), lambda i: (i, 0)),
            pl.BlockSpec((1, H), lambda i: (0, 0)),
            pl.BlockSpec((H, H), lambda i: (0, 0)),
            pl.BlockSpec((H, H), lambda i: (0, 0)),
            pl.BlockSpec((1, H), lambda i: (0, 0)),
        ],
        out_specs=[
            pl.BlockSpec((BLN, H), lambda i: (i, 0)),
            pl.BlockSpec((BLN, H), lambda i: (i, 0)),
        ],
        out_shape=[
            jax.ShapeDtypeStruct((NPAD, H), F32),
            jax.ShapeDtypeStruct((NPAD, H), F32),
        ],
    )(p0, p1, g, dinv, b.reshape(1, H), Wa, Wb, bb.reshape(1, H))


def _proj_img_body(f0_ref, f1_ref, wa0_ref, wa1_ref, wb0_ref, wb1_ref,
                   bb_ref, a_ref, bo_ref):
    f0 = f0_ref[...]
    f1 = f1_ref[...]
    a_ref[...] = (jnp.dot(f0, wa0_ref[...], preferred_element_type=F32)
                  + jnp.dot(f1, wa1_ref[...], preferred_element_type=F32))
    bo_ref[...] = (jnp.dot(f0, wb0_ref[...], preferred_element_type=F32)
                   + jnp.dot(f1, wb1_ref[...], preferred_element_type=F32)
                   + bb_ref[...])


def _tc_proj_img(F0, F1, liW, lib):
    wspec = pl.BlockSpec((H, H), lambda i: (0, 0))
    return pl.pallas_call(
        _proj_img_body,
        grid=(NPAD // BLN,),
        in_specs=[
            pl.BlockSpec((BLN, H), lambda i: (i, 0)),
            pl.BlockSpec((BLN, H), lambda i: (i, 0)),
            wspec, wspec, wspec, wspec,
            pl.BlockSpec((1, H), lambda i: (0, 0)),
        ],
        out_specs=[
            pl.BlockSpec((BLN, H), lambda i: (i, 0)),
            pl.BlockSpec((BLN, H), lambda i: (i, 0)),
        ],
        out_shape=[
            jax.ShapeDtypeStruct((NPAD, H), F32),
            jax.ShapeDtypeStruct((NPAD, H), F32),
        ],
    )(F0, F1, liW[0:H], liW[H:2 * H], liW[2 * H:3 * H], liW[3 * H:4 * H],
      lib.reshape(1, H))


def _conv_body(k_ref, x_ref, b_ref, o_ref):
    o_ref[...] = jnp.maximum(
        jnp.dot(k_ref[...], x_ref[...], preferred_element_type=F32)
        + b_ref[...], 0.0)


def _tc_conv(Km, X, b, sblk):
    O, KK = Km.shape
    S = X.shape[1]
    return pl.pallas_call(
        _conv_body,
        grid=(S // sblk,),
        in_specs=[
            pl.BlockSpec((O, KK), lambda i: (0, 0)),
            pl.BlockSpec((KK, sblk), lambda i: (0, i)),
            pl.BlockSpec((O, 1), lambda i: (0, 0)),
        ],
        out_specs=pl.BlockSpec((O, sblk), lambda i: (0, i)),
        out_shape=jax.ShapeDtypeStruct((O, S), F32),
    )(Km, X, b.reshape(O, 1))


BLG = 512


def _gru_body(xe_ref, wih_ref, whh_ref, bih_ref, bhh_ref, o_ref):
    h = jnp.zeros((BLG, H), F32)
    acc = jnp.zeros((BLG, H), F32)
    for t in range(L):
        xt = xe_ref[:, t * H:t * H + TD].astype(BF16)
        gi = jnp.dot(xt, wih_ref[...], preferred_element_type=F32) + bih_ref[...]
        gh = jnp.dot(h.astype(BF16), whh_ref[...],
                     preferred_element_type=F32) + bhh_ref[...]
        r = 1.0 / (1.0 + jnp.exp(-(gi[:, :H] + gh[:, :H])))
        z = 1.0 / (1.0 + jnp.exp(-(gi[:, H:2 * H] + gh[:, H:2 * H])))
        nc = jnp.tanh(gi[:, 2 * H:] + r * gh[:, 2 * H:])
        h = (1.0 - z) * nc + z * h
        acc = acc + h
    o_ref[...] = acc


def _tc_gru(xe2, WihT, WhhT, bih, bhh):
    return pl.pallas_call(
        _gru_body,
        grid=(NPAD // BLG,),
        in_specs=[
            pl.BlockSpec((BLG, L * H), lambda i: (i, 0)),
            pl.BlockSpec((TD, 3 * H), lambda i: (0, 0)),
            pl.BlockSpec((H, 3 * H), lambda i: (0, 0)),
            pl.BlockSpec((1, 3 * H), lambda i: (0, 0)),
            pl.BlockSpec((1, 3 * H), lambda i: (0, 0)),
        ],
        out_specs=pl.BlockSpec((BLG, H), lambda i: (i, 0)),
        out_shape=jax.ShapeDtypeStruct((NPAD, H), F32),
    )(xe2, WihT, WhhT, bih.reshape(1, 3 * H), bhh.reshape(1, 3 * H))


BLE = 1280


def _edge_mlp_body(a_ref, w1_ref, b1_ref, w2_ref, b2_ref, o_ref):
    a = a_ref[...].astype(BF16)
    u = jnp.maximum(
        jnp.dot(a, w1_ref[...], preferred_element_type=F32)
        + b1_ref[...], 0.0)
    v = jnp.dot(u.astype(BF16), w2_ref[...],
                preferred_element_type=F32) + b2_ref[...]
    v0 = v[:, 0:1]
    v1 = v[:, 1:2]
    m = jnp.maximum(v0, v1)
    lse = m + jnp.log(jnp.exp(v0 - m) + jnp.exp(v1 - m))
    o_ref[...] = jnp.concatenate([v0 - lse, v1 - lse], axis=1)


def _tc_edge_mlp(efeat, r1W, r1b, r2Wp, r2bp):
    es = efeat.shape[0]
    return pl.pallas_call(
        _edge_mlp_body,
        grid=(es // BLE,),
        in_specs=[
            pl.BlockSpec((BLE, 3 * H), lambda i: (i, 0)),
            pl.BlockSpec((3 * H, H), lambda i: (0, 0)),
            pl.BlockSpec((1, H), lambda i: (0, 0)),
            pl.BlockSpec((H, H), lambda i: (0, 0)),
            pl.BlockSpec((1, H), lambda i: (0, 0)),
        ],
        out_specs=pl.BlockSpec((BLE, 2), lambda i: (i, 0)),
        out_shape=jax.ShapeDtypeStruct((es, 2), F32),
    )(efeat, r1W, r1b.reshape(1, H), r2Wp, r2bp)


def _im2col(x3, stride=2):
    C, Hs, Ws = x3.shape
    Ho, Wo = Hs // stride, Ws // stride
    p = jnp.pad(x3, ((0, 0), (0, 1), (0, 1)))
    cols = []
    for ky in range(3):
        for kx in range(3):
            cols.append(lax.slice(p, (0, ky, kx),
                                  (C, ky + (Ho - 1) * stride + 1,
                                   kx + (Wo - 1) * stride + 1),
                                  (1, stride, stride)))
    return jnp.stack(cols, 0).reshape(9 * C, Ho * Wo)


def _kmat(K):
    O, C = K.shape[0], K.shape[1]
    return K.transpose(0, 2, 3, 1).reshape(O, 9 * C)


def kernel(x, edge_index, xtext, img, nodenum, pos, cell_wh, W1, b1, W2, b2,
           Wt1, bt1, Wt2, bt2, emb, gWih, gWhh, gbih, gbhh, K1, cb1, K2, cb2,
           K3, cb3, lpW, lpb, ltW, ltb, liW, lib, r1W, r1b, r2W, r2b):
    pad = NPAD - N
    src = edge_index[0].astype(I32)
    dst = edge_index[1].astype(I32)

    zer128 = jnp.zeros((RPT, H), F32)
    ones128 = jnp.ones((CH, H), F32)

    parts = _sc_deg(dst, ones128, zer128)
    dinv = _tc_dinv(parts)

    xp = jnp.pad(x, ((0, pad), (0, 0)))
    g1 = _tc_mm_scale(xp, W1, dinv)
    p1 = _sc_segsum(g1, src, dst, zer128)
    g2 = _tc_combine_mm_scale(p1[0], p1[1], g1, dinv, b1, W2)
    p2 = _sc_segsum(g2, src, dst, zer128)
    Ap, Bp = _tc_combine_proj(p2[0], p2[1], g2, dinv, b2,
                              lpW[:H], lpW[H:], lpb)

    xtp = jnp.pad(xtext.astype(I32), ((0, pad), (0, 0))).reshape(-1)
    embp = jnp.pad(emb, ((0, 0), (0, H - TD)))
    xe = _sc_embgather(embp, xtp)
    xe2 = xe.reshape(NPAD, L * H)
    tf0 = _tc_gru(xe2, gWih.T.astype(BF16), gWhh.T.astype(BF16), gbih, gbhh)
    gt1 = _tc_mm_scale(tf0, Wt1, dinv)
    pt1 = _sc_segsum(gt1, src, dst, zer128)
    gt2 = _tc_combine_mm_scale(pt1[0], pt1[1], gt1, dinv, bt1, Wt2)
    pt2 = _sc_segsum(gt2, src, dst, zer128)
    At, Bt = _tc_combine_proj(pt2[0], pt2[1], gt2, dinv, bt2,
                              ltW[:H], ltW[H:], ltb)

    X1 = jnp.pad(_im2col(img[0]), ((0, 5), (0, 0)))
    Km1 = jnp.pad(_kmat(K1), ((0, 0), (0, 5)))
    f1 = _tc_conv(Km1, X1, cb1, 2048)
    X2 = _im2col(f1.reshape(32, 256, 256))
    f2 = _tc_conv(_kmat(K2), X2, cb2, 2048)
    X3 = _im2col(f2.reshape(64, 128, 128))
    fm = _tc_conv(_kmat(K3), X3, cb3, 2048)
    fmT = fm.T

    posp = jnp.pad(pos, ((0, pad), (0, 0)))
    cwp = jnp.pad(cell_wh, ((0, pad), (0, 0)))
    Fb = _sc_boxgather(fmT, posp[:, 0], posp[:, 1], cwp[:, 0], cwp[:, 1])
    Ai, Bi = _tc_proj_img(Fb[0], Fb[1], liW, lib)

    SrcTab = jnp.concatenate([Ap, At, Ai], axis=1)
    DstTab = jnp.concatenate([Bp, Bt, Bi], axis=1)
    r1Wb = r1W.astype(BF16)
    r2Wp = jnp.pad(r2W, ((0, 0), (0, H - 2))).astype(BF16)
    r2bp = jnp.pad(r2b, (0, H - 2)).reshape(1, H)
    outs = []
    for k in range(NSLAB):
        sl = slice(k * ES, (k + 1) * ES)
        ef = _sc_edgefeat(SrcTab, DstTab, src[sl], dst[sl])
        outs.append(_tc_edge_mlp(ef, r1Wb, r1b, r2Wp, r2bp))
    return jnp.concatenate(outs, axis=0)


# speedup vs baseline: 1.0039x; 1.0039x over previous
import functools

import jax
import jax.numpy as jnp
from jax import lax
from jax.experimental import pallas as pl
from jax.experimental.pallas import tpu as pltpu
from jax.experimental.pallas import tpu_sc as plsc

F32 = jnp.float32
I32 = jnp.int32
BF16 = jnp.bfloat16

N = 10000
E = 320000
H = 128
TD = 64
L = 16
NPAD = 10240
NSC = 2
NSUB = 16
NW = NSC * NSUB
CH = 80
RPT = NPAD // NSUB

_mesh = plsc.VectorSubcoreMesh(core_axis_name="c", subcore_axis_name="s")
_NCHUNK = (E // NW) // CH
_NCPAD = 128


def _wid():
    return lax.axis_index("s") * NSC + lax.axis_index("c")


@functools.partial(
    pl.kernel, mesh=_mesh,
    out_type=jax.ShapeDtypeStruct((NSC, NPAD, H), F32),
    scratch_types=[
        pltpu.VMEM((CH,), I32),
        pltpu.VMEM((CH,), I32),
        pltpu.VMEM((CH, H), F32),
        pltpu.VMEM_SHARED((NPAD, H), F32),
    ],
)
def _sc_deg(dst_hbm, ones_hbm, zer_hbm, out_hbm, dstv0, dstv1, onesv, acc):
    c = lax.axis_index("c")
    s = lax.axis_index("s")
    wid = _wid()
    dstv = (dstv0, dstv1)
    pltpu.sync_copy(zer_hbm, acc.at[pl.ds(s * RPT, RPT), :])
    pltpu.sync_copy(ones_hbm, onesv)
    plsc.subcore_barrier()
    base = wid * (E // NW)

    def issue(ci, b):
        pltpu.sync_copy(dst_hbm.at[pl.ds(base + ci * CH, CH)], dstv[b])

    def drain(b):
        pltpu.sync_copy(onesv, acc.at[dstv[b]], add=True)

    issue(0, 0)

    def body(ci, _):
        issue(2 * ci + 1, 1)
        drain(0)
        issue(2 * ci + 2, 0)
        drain(1)
        return 0

    lax.fori_loop(0, (_NCHUNK - 1) // 2, body, 0)
    drain(0)
    plsc.subcore_barrier()
    pltpu.sync_copy(acc.at[pl.ds(s * RPT, RPT), :],
                    out_hbm.at[c, pl.ds(s * RPT, RPT), :])


@functools.partial(
    pl.kernel, mesh=_mesh,
    out_type=jax.ShapeDtypeStruct((NSC, NPAD, H), F32),
    scratch_types=[
        pltpu.VMEM((E // NW,), I32),
        pltpu.VMEM((CH,), I32),
        pltpu.VMEM((CH,), I32),
        pltpu.VMEM((CH, H), F32),
        pltpu.VMEM((CH, H), F32),
        pltpu.VMEM_SHARED((NPAD, H), F32),
        pltpu.SemaphoreType.DMA,
        pltpu.SemaphoreType.DMA,
    ],
)
def _sc_segsum(g_hbm, src_hbm, dst_hbm, zer_hbm, out_hbm,
               srcall, dstv0, dstv1, rows0, rows1, acc, sem0, sem1):
    c = lax.axis_index("c")
    s = lax.axis_index("s")
    wid = _wid()
    dstv = (dstv0, dstv1)
    rows = (rows0, rows1)
    sem = (sem0, sem1)
    epw = E // NW
    base = wid * epw
    pltpu.sync_copy(zer_hbm, acc.at[pl.ds(s * RPT, RPT), :])
    pltpu.sync_copy(src_hbm.at[pl.ds(base, epw)], srcall)
    plsc.subcore_barrier()

    def issue(ci, b):
        pltpu.sync_copy(dst_hbm.at[pl.ds(base + ci * CH, CH)], dstv[b])
        pltpu.async_copy(g_hbm.at[srcall.at[pl.ds(ci * CH, CH)]],
                         rows[b], sem[b])

    def drain(ci, b):
        pltpu.make_async_copy(g_hbm.at[srcall.at[pl.ds(ci * CH, CH)]],
                              rows[b], sem[b]).wait()
        pltpu.sync_copy(rows[b], acc.at[dstv[b]], add=True)

    issue(0, 0)

    def body(ci, _):
        issue(2 * ci + 1, 1)
        drain(2 * ci, 0)
        issue(2 * ci + 2, 0)
        drain(2 * ci + 1, 1)
        return 0

    lax.fori_loop(0, (_NCHUNK - 1) // 2, body, 0)
    drain(_NCHUNK - 1, 0)
    plsc.subcore_barrier()
    pltpu.sync_copy(acc.at[pl.ds(s * RPT, RPT), :],
                    out_hbm.at[c, pl.ds(s * RPT, RPT), :])


@functools.partial(
    pl.kernel, mesh=_mesh,
    out_type=jax.ShapeDtypeStruct((NPAD * L, H), F32),
    scratch_types=[
        pltpu.VMEM((CH,), I32),
        pltpu.VMEM((CH,), I32),
        pltpu.VMEM((CH, H), F32),
        pltpu.VMEM((CH, H), F32),
        pltpu.SemaphoreType.DMA,
        pltpu.SemaphoreType.DMA,
    ],
)
def _sc_embgather(tab_hbm, idx_hbm, out_hbm, idxv0, idxv1, rows0, rows1,
                  sem0, sem1):
    wid = _wid()
    per = (NPAD * L) // NW
    base = wid * per
    nchunk = per // CH
    idxv = (idxv0, idxv1)
    rows = (rows0, rows1)
    sem = (sem0, sem1)

    def issue(ci, b):
        off = base + ci * CH
        pltpu.sync_copy(idx_hbm.at[pl.ds(off, CH)], idxv[b])
        pltpu.async_copy(tab_hbm.at[idxv[b]], rows[b], sem[b])

    def drain(ci, b):
        pltpu.make_async_copy(tab_hbm.at[idxv[b]], rows[b], sem[b]).wait()
        off = base + ci * CH
        pltpu.sync_copy(rows[b], out_hbm.at[pl.ds(off, CH), :])

    issue(0, 0)

    def body(ci, _):
        issue(2 * ci + 1, 1)
        drain(2 * ci, 0)
        issue(2 * ci + 2, 0)
        drain(2 * ci + 1, 1)
        return 0

    lax.fori_loop(0, nchunk // 2 - 1, body, 0)
    issue(nchunk - 1, 1)
    drain(nchunk - 2, 0)
    drain(nchunk - 1, 1)


@functools.partial(
    pl.kernel, mesh=_mesh,
    out_type=jax.ShapeDtypeStruct((2, NPAD, H), F32),
    scratch_types=[
        pltpu.VMEM((NPAD // NW,), F32),
        pltpu.VMEM((NPAD // NW,), F32),
        pltpu.VMEM((NPAD // NW,), F32),
        pltpu.VMEM((NPAD // NW,), F32),
        pltpu.VMEM((CH,), I32),
        pltpu.VMEM((CH,), I32),
        pltpu.VMEM((CH, H), F32),
        pltpu.SemaphoreType.DMA,
    ],
)
def _sc_boxgather(fmT_hbm, px_hbm, py_hbm, cx_hbm, cy_hbm, out_hbm,
                  pxv, pyv, cxv, cyv, idx0, idx1, rows, sem):
    wid = _wid()
    per = NPAD // NW
    base = wid * per
    pltpu.sync_copy(px_hbm.at[pl.ds(base, per)], pxv)
    pltpu.sync_copy(py_hbm.at[pl.ds(base, per)], pyv)
    pltpu.sync_copy(cx_hbm.at[pl.ds(base, per)], cxv)
    pltpu.sync_copy(cy_hbm.at[pl.ds(base, per)], cyv)

    def body(ci, _):
        off = base + ci * CH
        for j in range(CH // 16):
            cs = pl.ds(ci * CH + j * 16, 16)
            sl = pl.ds(j * 16, 16)
            vx = pxv[cs]
            vy = pyv[cs]
            wx = cxv[cs]
            wy = cyv[cs]
            for snum, tt in ((0, -0.25), (1, 0.25)):
                qx = jnp.minimum(jnp.maximum(vx + tt * wx, 0.0), 1.0) * 64.0
                qy = jnp.minimum(jnp.maximum(vy + tt * wy, 0.0), 1.0) * 64.0
                ix = jnp.minimum(jnp.maximum(qx.astype(I32), 0), 63)
                iy = jnp.minimum(jnp.maximum(qy.astype(I32), 0), 63)
                lin = iy * 64 + ix
                if snum == 0:
                    idx0[sl] = lin
                else:
                    idx1[sl] = lin
        pltpu.async_copy(fmT_hbm.at[idx0], rows, sem).wait()
        pltpu.sync_copy(rows, out_hbm.at[0, pl.ds(off, CH), :])
        pltpu.async_copy(fmT_hbm.at[idx1], rows, sem).wait()
        pltpu.sync_copy(rows, out_hbm.at[1, pl.ds(off, CH), :])
        return 0

    lax.fori_loop(0, per // CH, body, 0)


EFW = 3 * H
NSLAB = 10
ES = E // NSLAB
ESCH = 40


def _make_edgefeat(es, ch=CH):
    nchunk = (es // NW) // ch

    @functools.partial(
        pl.kernel, mesh=_mesh,
        out_type=jax.ShapeDtypeStruct((es, EFW), F32),
        scratch_types=[
            pltpu.VMEM((es // NW,), I32),
            pltpu.VMEM((es // NW,), I32),
            pltpu.VMEM((ch, EFW), F32),
            pltpu.VMEM((ch, EFW), F32),
            pltpu.VMEM((ch, EFW), F32),
            pltpu.VMEM((ch, EFW), F32),
            pltpu.SemaphoreType.DMA,
            pltpu.SemaphoreType.DMA,
            pltpu.SemaphoreType.DMA,
            pltpu.SemaphoreType.DMA,
        ],
    )
    def _ef(stab_hbm, dtab_hbm, src_hbm, dst_hbm, out_hbm,
            srcall, dstall, ar0, br0, ar1, br1,
            sa0, sb0, sa1, sb1):
        wid = _wid()
        epw = es // NW
        base = wid * epw
        ar = (ar0, ar1)
        br = (br0, br1)
        sa = (sa0, sa1)
        sb = (sb0, sb1)
        pltpu.sync_copy(src_hbm.at[pl.ds(base, epw)], srcall)
        pltpu.sync_copy(dst_hbm.at[pl.ds(base, epw)], dstall)

        def issue(ci, b):
            sl = pl.ds(ci * ch, ch)
            pltpu.async_copy(stab_hbm.at[srcall.at[sl]], ar[b], sa[b])
            pltpu.async_copy(dtab_hbm.at[dstall.at[sl]], br[b], sb[b])

        def drain(ci, b):
            sl = pl.ds(ci * ch, ch)
            pltpu.make_async_copy(stab_hbm.at[srcall.at[sl]], ar[b], sa[b]).wait()
            pltpu.make_async_copy(dtab_hbm.at[dstall.at[sl]], br[b], sb[b]).wait()
            arb = ar[b]
            brb = br[b]

            def rowbody(i, _):
                for j in range(EFW // 16):
                    sl = pl.ds(j * 16, 16)
                    arb[i, sl] = jnp.maximum(arb[i, sl] + brb[i, sl], 0.0)
                return 0

            lax.fori_loop(0, ch, rowbody, 0)
            off = base + ci * ch
            pltpu.sync_copy(arb, out_hbm.at[pl.ds(off, ch), :])

        issue(0, 0)

        def body(ci, _):
            issue(2 * ci + 1, 1)
            drain(2 * ci, 0)
            issue(2 * ci + 2, 0)
            drain(2 * ci + 1, 1)
            return 0

        lax.fori_loop(0, (nchunk - 1) // 2, body, 0)
        drain(nchunk - 1, 0)

    return _ef


_sc_edgefeat = _make_edgefeat(ES, ESCH)


def _dinv_body(p_ref, o_ref):
    s = p_ref[0] + p_ref[1]
    o_ref[...] = lax.rsqrt(s[:, 0:1] + 1.0)


def _tc_dinv(parts):
    BB = 2048
    return pl.pallas_call(
        _dinv_body,
        grid=(NPAD // BB,),
        in_specs=[pl.BlockSpec((NSC, BB, H), lambda i: (0, i, 0))],
        out_specs=pl.BlockSpec((BB, 1), lambda i: (i, 0)),
        out_shape=jax.ShapeDtypeStruct((NPAD, 1), F32),
    )(parts)


BLN = 1024


def _mm_scale_body(x_ref, w_ref, dv_ref, o_ref):
    o_ref[...] = jnp.dot(x_ref[...], w_ref[...],
                         preferred_element_type=F32) * dv_ref[...]


def _tc_mm_scale(x, W, dinv):
    n = x.shape[0]
    return pl.pallas_call(
        _mm_scale_body,
        grid=(n // BLN,),
        in_specs=[
            pl.BlockSpec((BLN, x.shape[1]), lambda i: (i, 0)),
            pl.BlockSpec(W.shape, lambda i: (0, 0)),
            pl.BlockSpec((BLN, 1), lambda i: (i, 0)),
        ],
        out_specs=pl.BlockSpec((BLN, H), lambda i: (i, 0)),
        out_shape=jax.ShapeDtypeStruct((n, H), F32),
    )(x, W, dinv)


def _comb_mm_body(p0_ref, p1_ref, g_ref, dv_ref, b_ref, w_ref, o_ref):
    t = jnp.maximum(dv_ref[...] * (p0_ref[...] + p1_ref[...] + g_ref[...])
                    + b_ref[...], 0.0)
    o_ref[...] = jnp.dot(t, w_ref[...], preferred_element_type=F32) * dv_ref[...]


def _tc_combine_mm_scale(p0, p1, g, dinv, b, W):
    return pl.pallas_call(
        _comb_mm_body,
        grid=(NPAD // BLN,),
        in_specs=[
            pl.BlockSpec((BLN, H), lambda i: (i, 0)),
            pl.BlockSpec((BLN, H), lambda i: (i, 0)),
            pl.BlockSpec((BLN, H), lambda i: (i, 0)),
            pl.BlockSpec((BLN, 1), lambda i: (i, 0)),
            pl.BlockSpec((1, H), lambda i: (0, 0)),
            pl.BlockSpec((H, H), lambda i: (0, 0)),
        ],
        out_specs=pl.BlockSpec((BLN, H), lambda i: (i, 0)),
        out_shape=jax.ShapeDtypeStruct((NPAD, H), F32),
    )(p0, p1, g, dinv, b.reshape(1, H), W)


def _comb_proj_body(p0_ref, p1_ref, g_ref, dv_ref, b_ref, wa_ref, wb_ref,
                    bb_ref, a_ref, bo_ref):
    t = jnp.maximum(dv_ref[...] * (p0_ref[...] + p1_ref[...] + g_ref[...])
                    + b_ref[...], 0.0)
    a_ref[...] = jnp.dot(t, wa_ref[...], preferred_element_type=F32)
    bo_ref[...] = jnp.dot(t, wb_ref[...], preferred_element_type=F32) + bb_ref[...]


def _tc_combine_proj(p0, p1, g, dinv, b, Wa, Wb, bb):
    return pl.pallas_call(
        _comb_proj_body,
        grid=(NPAD // BLN,),
        in_specs=[
            pl.BlockSpec((BLN, H), lambda i: (i, 0)),
            pl.BlockSpec((BLN, H), lambda i: (i, 0)),
            pl.BlockSpec((BLN, H), lambda i: (i, 0)),
            pl.BlockSpec((BLN, 1), lambda i: (i, 0)),
            pl.BlockSpec((1, H), lambda i: (0, 0)),
            pl.BlockSpec((H, H), lambda i: (0, 0)),
            pl.BlockSpec((H, H), lambda i: (0, 0)),
            pl.BlockSpec((1, H), lambda i: (0, 0)),
        ],
        out_specs=[
            pl.BlockSpec((BLN, H), lambda i: (i, 0)),
            pl.BlockSpec((BLN, H), lambda i: (i, 0)),
        ],
        out_shape=[
            jax.ShapeDtypeStruct((NPAD, H), F32),
            jax.ShapeDtypeStruct((NPAD, H), F32),
        ],
    )(p0, p1, g, dinv, b.reshape(1, H), Wa, Wb, bb.reshape(1, H))


def _proj_img_body(f0_ref, f1_ref, wa0_ref, wa1_ref, wb0_ref, wb1_ref,
                   bb_ref, a_ref, bo_ref):
    f0 = f0_ref[...]
    f1 = f1_ref[...]
    a_ref[...] = (jnp.dot(f0, wa0_ref[...], preferred_element_type=F32)
                  + jnp.dot(f1, wa1_ref[...], preferred_element_type=F32))
    bo_ref[...] = (jnp.dot(f0, wb0_ref[...], preferred_element_type=F32)
                   + jnp.dot(f1, wb1_ref[...], preferred_element_type=F32)
                   + bb_ref[...])


def _tc_proj_img(F0, F1, liW, lib):
    wspec = pl.BlockSpec((H, H), lambda i: (0, 0))
    return pl.pallas_call(
        _proj_img_body,
        grid=(NPAD // BLN,),
        in_specs=[
            pl.BlockSpec((BLN, H), lambda i: (i, 0)),
            pl.BlockSpec((BLN, H), lambda i: (i, 0)),
            wspec, wspec, wspec, wspec,
            pl.BlockSpec((1, H), lambda i: (0, 0)),
        ],
        out_specs=[
            pl.BlockSpec((BLN, H), lambda i: (i, 0)),
            pl.BlockSpec((BLN, H), lambda i: (i, 0)),
        ],
        out_shape=[
            jax.ShapeDtypeStruct((NPAD, H), F32),
            jax.ShapeDtypeStruct((NPAD, H), F32),
        ],
    )(F0, F1, liW[0:H], liW[H:2 * H], liW[2 * H:3 * H], liW[3 * H:4 * H],
      lib.reshape(1, H))


def _conv_body(k_ref, x_ref, b_ref, o_ref):
    o_ref[...] = jnp.maximum(
        jnp.dot(k_ref[...], x_ref[...], preferred_element_type=F32)
        + b_ref[...], 0.0)


def _tc_conv(Km, X, b, sblk):
    O, KK = Km.shape
    S = X.shape[1]
    return pl.pallas_call(
        _conv_body,
        grid=(S // sblk,),
        in_specs=[
            pl.BlockSpec((O, KK), lambda i: (0, 0)),
            pl.BlockSpec((KK, sblk), lambda i: (0, i)),
            pl.BlockSpec((O, 1), lambda i: (0, 0)),
        ],
        out_specs=pl.BlockSpec((O, sblk), lambda i: (0, i)),
        out_shape=jax.ShapeDtypeStruct((O, S), F32),
    )(Km, X, b.reshape(O, 1))


BLG = 512


def _gru_body(xe_ref, wih_ref, whh_ref, bih_ref, bhh_ref, o_ref):
    h = jnp.zeros((BLG, H), F32)
    acc = jnp.zeros((BLG, H), F32)
    for t in range(L):
        xt = xe_ref[:, t * H:t * H + TD].astype(BF16)
        gi = jnp.dot(xt, wih_ref[...], preferred_element_type=F32) + bih_ref[...]
        gh = jnp.dot(h.astype(BF16), whh_ref[...],
                     preferred_element_type=F32) + bhh_ref[...]
        r = 1.0 / (1.0 + jnp.exp(-(gi[:, :H] + gh[:, :H])))
        z = 1.0 / (1.0 + jnp.exp(-(gi[:, H:2 * H] + gh[:, H:2 * H])))
        nc = jnp.tanh(gi[:, 2 * H:] + r * gh[:, 2 * H:])
        h = (1.0 - z) * nc + z * h
        acc = acc + h
    o_ref[...] = acc


def _tc_gru(xe2, WihT, WhhT, bih, bhh):
    return pl.pallas_call(
        _gru_body,
        grid=(NPAD // BLG,),
        in_specs=[
            pl.BlockSpec((BLG, L * H), lambda i: (i, 0)),
            pl.BlockSpec((TD, 3 * H), lambda i: (0, 0)),
            pl.BlockSpec((H, 3 * H), lambda i: (0, 0)),
            pl.BlockSpec((1, 3 * H), lambda i: (0, 0)),
            pl.BlockSpec((1, 3 * H), lambda i: (0, 0)),
        ],
        out_specs=pl.BlockSpec((BLG, H), lambda i: (i, 0)),
        out_shape=jax.ShapeDtypeStruct((NPAD, H), F32),
    )(xe2, WihT, WhhT, bih.reshape(1, 3 * H), bhh.reshape(1, 3 * H))


BLE = 1280


def _edge_mlp_body(a_ref, w1_ref, b1_ref, w2_ref, b2_ref, o_ref):
    a = a_ref[...].astype(BF16)
    u = jnp.maximum(
        jnp.dot(a, w1_ref[...], preferred_element_type=F32)
        + b1_ref[...], 0.0)
    v = jnp.dot(u.astype(BF16), w2_ref[...],
                preferred_element_type=F32) + b2_ref[...]
    v0 = v[:, 0:1]
    v1 = v[:, 1:2]
    m = jnp.maximum(v0, v1)
    lse = m + jnp.log(jnp.exp(v0 - m) + jnp.exp(v1 - m))
    o_ref[...] = jnp.concatenate([v0 - lse, v1 - lse], axis=1)


def _tc_edge_mlp(efeat, r1W, r1b, r2Wp, r2bp):
    es = efeat.shape[0]
    return pl.pallas_call(
        _edge_mlp_body,
        grid=(es // BLE,),
        in_specs=[
            pl.BlockSpec((BLE, 3 * H), lambda i: (i, 0)),
            pl.BlockSpec((3 * H, H), lambda i: (0, 0)),
            pl.BlockSpec((1, H), lambda i: (0, 0)),
            pl.BlockSpec((H, H), lambda i: (0, 0)),
            pl.BlockSpec((1, H), lambda i: (0, 0)),
        ],
        out_specs=pl.BlockSpec((BLE, 2), lambda i: (i, 0)),
        out_shape=jax.ShapeDtypeStruct((es, 2), F32),
    )(efeat, r1W, r1b.reshape(1, H), r2Wp, r2bp)


def _im2col(x3, stride=2):
    C, Hs, Ws = x3.shape
    Ho, Wo = Hs // stride, Ws // stride
    p = jnp.pad(x3, ((0, 0), (0, 1), (0, 1)))
    cols = []
    for ky in range(3):
        for kx in range(3):
            cols.append(lax.slice(p, (0, ky, kx),
                                  (C, ky + (Ho - 1) * stride + 1,
                                   kx + (Wo - 1) * stride + 1),
                                  (1, stride, stride)))
    return jnp.stack(cols, 0).reshape(9 * C, Ho * Wo)


def _kmat(K):
    O, C = K.shape[0], K.shape[1]
    return K.transpose(0, 2, 3, 1).reshape(O, 9 * C)


def kernel(x, edge_index, xtext, img, nodenum, pos, cell_wh, W1, b1, W2, b2,
           Wt1, bt1, Wt2, bt2, emb, gWih, gWhh, gbih, gbhh, K1, cb1, K2, cb2,
           K3, cb3, lpW, lpb, ltW, ltb, liW, lib, r1W, r1b, r2W, r2b):
    pad = NPAD - N
    src = edge_index[0].astype(I32)
    dst = edge_index[1].astype(I32)

    zer128 = jnp.zeros((RPT, H), F32)
    ones128 = jnp.ones((CH, H), F32)

    parts = _sc_deg(dst, ones128, zer128)
    dinv = _tc_dinv(parts)

    xp = jnp.pad(x, ((0, pad), (0, 0)))
    g1 = _tc_mm_scale(xp, W1, dinv)
    p1 = _sc_segsum(g1, src, dst, zer128)
    g2 = _tc_combine_mm_scale(p1[0], p1[1], g1, dinv, b1, W2)
    p2 = _sc_segsum(g2, src, dst, zer128)
    Ap, Bp = _tc_combine_proj(p2[0], p2[1], g2, dinv, b2,
                              lpW[:H], lpW[H:], lpb)

    xtp = jnp.pad(xtext.astype(I32), ((0, pad), (0, 0))).reshape(-1)
    embp = jnp.pad(emb, ((0, 0), (0, H - TD)))
    xe = _sc_embgather(embp, xtp)
    xe2 = xe.reshape(NPAD, L * H)
    tf0 = _tc_gru(xe2, gWih.T.astype(BF16), gWhh.T.astype(BF16), gbih, gbhh)
    gt1 = _tc_mm_scale(tf0, Wt1, dinv)
    pt1 = _sc_segsum(gt1, src, dst, zer128)
    gt2 = _tc_combine_mm_scale(pt1[0], pt1[1], gt1, dinv, bt1, Wt2)
    pt2 = _sc_segsum(gt2, src, dst, zer128)
    At, Bt = _tc_combine_proj(pt2[0], pt2[1], gt2, dinv, bt2,
                              ltW[:H], ltW[H:], ltb)

    X1 = jnp.pad(_im2col(img[0]), ((0, 5), (0, 0)))
    Km1 = jnp.pad(_kmat(K1), ((0, 0), (0, 5)))
    f1 = _tc_conv(Km1, X1, cb1, 2048)
    X2 = _im2col(f1.reshape(32, 256, 256))
    f2 = _tc_conv(_kmat(K2), X2, cb2, 2048)
    X3 = _im2col(f2.reshape(64, 128, 128))
    fm = _tc_conv(_kmat(K3), X3, cb3, 2048)
    fmT = fm.T

    posp = jnp.pad(pos, ((0, pad), (0, 0)))
    cwp = jnp.pad(cell_wh, ((0, pad), (0, 0)))
    Fb = _sc_boxgather(fmT, posp[:, 0], posp[:, 1], cwp[:, 0], cwp[:, 1])
    Ai, Bi = _tc_proj_img(Fb[0], Fb[1], liW, lib)

    SrcTab = jnp.concatenate([Ap, At, Ai], axis=1)
    DstTab = jnp.concatenate([Bp, Bt, Bi], axis=1)
    r1Wb = r1W.astype(BF16)
    r2Wp = jnp.pad(r2W, ((0, 0), (0, H - 2))).astype(BF16)
    r2bp = jnp.pad(r2b, (0, H - 2)).reshape(1, H)
    outs = []
    for k in range(NSLAB):
        sl = slice(k * ES, (k + 1) * ES)
        ef = _sc_edgefeat(SrcTab, DstTab, src[sl], dst[sl])
        outs.append(_tc_edge_mlp(ef, r1Wb, r1b, r2Wp, r2bp))
    return jnp.concatenate(outs, axis=0)
